# baseline (speedup 1.0000x reference)
import functools
import math

import jax
import jax.numpy as jnp
import numpy as np
from jax.experimental import pallas as pl
from jax.experimental.pallas import tpu as pltpu

IMG = 512
STEPS = (4, 8, 16, 32)
SCALES = (0.04, 0.1, 0.26, 0.45, 0.58)
FHW = (128, 64, 32, 16)
CH = (96, 192, 384, 768)
NPIX = tuple(f * f for f in FHW)
PBLK = 256
NBLK = tuple(n // PBLK for n in NPIX)
BLK_START = (0, 64, 80, 84)
NBLK_ALL = 85
NPIX_ALL = 21760
NB = 4
NCONF = NB * 2
NLOC = NB * 4
NCH = NCONF + NLOC


def _anchor_const_table() -> np.ndarray:
    tab = np.zeros((4, NB * 14), dtype=np.float32)
    for i in range(4):
        s, sn = SCALES[i], SCALES[i + 1]
        wh = [
            (s, s),
            (math.sqrt(s * sn), math.sqrt(s * sn)),
            (s * math.sqrt(2.0), s / math.sqrt(2.0)),
            (s * math.sqrt(0.5), s / math.sqrt(0.5)),
        ]
        for k in range(NB):
            base = 14 * k
            tab[i, base + 8] = wh[k][0]
            tab[i, base + 9] = wh[k][1]
            tab[i, base + 10:base + 14] = (0.1, 0.1, 0.2, 0.2)
    return tab


_CONST56 = _anchor_const_table()


def _ssd_head_kernel(f0, f1, f2, f3, w0, w1, w2, w3, btab, ctab, out_ref,
                     scratch):
    j = pl.program_id(1)
    feats = (f0, f1, f2, f3)
    ws = (w0, w1, w2, w3)

    for i in range(4):
        fw = FHW[i]
        npix = NPIX[i]
        start = BLK_START[i]
        in_level = (j >= start) & (j < start + NBLK[i])

        @pl.when(in_level & (j == start))
        def _conv(i=i, fw=fw, npix=npix):
            fh = fw
            w = ws[i][...]
            bias = jnp.broadcast_to(btab[i:i + 1, 0:NCH], (fw, NCH))
            nchunk = 4 if i == 0 else 1
            cr = fh // nchunk
            for c in range(nchunk):
                r0, r1 = c * cr, (c + 1) * cr
                lo, hi = max(0, r0 - 1), min(fh, r1 + 1)
                x = feats[i][0, lo:hi].reshape((hi - lo) * fw, CH[i])
                t = jnp.dot(x, w, preferred_element_type=jnp.float32)
                t3 = t.reshape(hi - lo, fw, 9 * NCH)
                scratch[r0:r1, 0:fw, :] = jnp.broadcast_to(
                    bias[None], (r1 - r0, fw, NCH))
                for ky in range(3):
                    for kx in range(3):
                        oy, ox = ky - 1, kx - 1
                        q = ky * 3 + kx
                        y0 = max(r0, -oy)
                        y1 = min(r1, fh - oy)
                        x0 = max(0, -ox)
                        x1 = fw - max(0, ox)
                        term = t3[y0 + oy - lo:y1 + oy - lo,
                                  x0 + ox:x1 + ox,
                                  q * NCH:(q + 1) * NCH]
                        scratch[y0:y1, x0:x1, :] += term

        @pl.when(in_level)
        def _emit(i=i, fw=fw, start=start):
            lb = j - start
            nr = PBLK // fw
            rows = scratch[pl.ds(lb * nr, nr), 0:fw, :].reshape(PBLK, NCH)
            conf = rows[:, 0:NCONF]
            locv = rows[:, NCONF:NCH]
            lane = jax.lax.broadcasted_iota(jnp.int32, (PBLK, NCONF), 1)
            swapped = jnp.where(jnp.bitwise_and(lane, 1) == 0,
                                jnp.roll(conf, -1, axis=1),
                                jnp.roll(conf, 1, axis=1))
            p8 = jax.nn.sigmoid(conf - swapped)
            pix = lb * PBLK + jax.lax.broadcasted_iota(jnp.int32, (PBLK, 1), 0)
            xcol = jnp.bitwise_and(pix, fw - 1)
            yrow = jax.lax.shift_right_logical(pix, int(math.log2(fw)))
            scale = float(STEPS[i]) / float(IMG)
            cx = (xcol.astype(jnp.float32) + 0.5) * scale
            cy = (yrow.astype(jnp.float32) + 0.5) * scale
            pieces = []
            for k in range(NB):
                b = 14 * k
                pieces.append(p8[:, 2 * k:2 * k + 2])
                pieces.append(locv[:, 4 * k:4 * k + 4])
                pieces.append(cx)
                pieces.append(cy)
                pieces.append(jnp.broadcast_to(ctab[i:i + 1, b + 8:b + 14],
                                               (PBLK, 6)))
            out_ref[0] = jnp.concatenate(pieces, axis=1)


def kernel(feat0, feat1, feat2, feat3, Wc0, bc0, Wl0, bl0, Wc1, bc1, Wl1,
           bl1, Wc2, bc2, Wl2, bl2, Wc3, bc3, Wl3, bl3):
    B = feat0.shape[0]
    feats = (feat0, feat1, feat2, feat3)
    Wc = (Wc0, Wc1, Wc2, Wc3)
    bc = (bc0, bc1, bc2, bc3)
    Wl = (Wl0, Wl1, Wl2, Wl3)
    bl = (bl0, bl1, bl2, bl3)

    w_all = []
    for i in range(4):
        blocks = []
        for ky in range(3):
            for kx in range(3):
                blocks.append(Wc[i][ky, kx])
                blocks.append(Wl[i][ky, kx])
        w_all.append(jnp.concatenate(blocks, axis=-1))

    btab = jnp.zeros((8, 128), jnp.float32)
    for i in range(4):
        btab = btab.at[i, 0:NCONF].set(bc[i])
        btab = btab.at[i, NCONF:NCH].set(bl[i])
    ctab = jnp.zeros((8, 128), jnp.float32)
    ctab = ctab.at[0:4, 0:NB * 14].set(jnp.asarray(_CONST56))

    def feat_spec(i):
        s = BLK_START[i]
        return pl.BlockSpec(
            (1, FHW[i], FHW[i], CH[i]),
            lambda b, j, s=s: (jnp.minimum(b + (j > s).astype(jnp.int32),
                                           B - 1), 0, 0, 0))

    def whole(arr):
        return pl.BlockSpec(arr.shape, lambda b, j: (0,) * arr.ndim)

    out = pl.pallas_call(
        _ssd_head_kernel,
        grid=(B, NBLK_ALL),
        in_specs=[feat_spec(i) for i in range(4)]
        + [whole(w) for w in w_all] + [whole(btab), whole(ctab)],
        out_specs=pl.BlockSpec((1, PBLK, NB * 14), lambda b, j: (b, j, 0)),
        out_shape=jax.ShapeDtypeStruct((B, NPIX_ALL, NB * 14), jnp.float32),
        scratch_shapes=[pltpu.VMEM((FHW[0], FHW[0], NCH), jnp.float32)],
        compiler_params=pltpu.CompilerParams(
            dimension_semantics=("arbitrary", "arbitrary"),
            vmem_limit_bytes=128 * 1024 * 1024,
        ),
    )(*feats, *w_all, btab, ctab)
    return out.reshape(B, NPIX_ALL * NB, 14)


# speedup vs baseline: 1.0752x; 1.0752x over previous
import functools
import math

import jax
import jax.numpy as jnp
import numpy as np
from jax.experimental import pallas as pl
from jax.experimental.pallas import tpu as pltpu

IMG = 512
STEPS = (4, 8, 16, 32)
SCALES = (0.04, 0.1, 0.26, 0.45, 0.58)
FHW = (128, 64, 32, 16)
CH = (96, 192, 384, 768)
NPIX = tuple(f * f for f in FHW)
PBLK = 1024
NBLK = (16, 4, 1, 1)
BLK_START = (0, 16, 20, 21)
NBLK_ALL = 22
NPIX_ALL = 21760
NB = 4
NCONF = NB * 2
NLOC = NB * 4
NCH = NCONF + NLOC


def _anchor_const_table() -> np.ndarray:
    tab = np.zeros((4, NB * 14), dtype=np.float32)
    for i in range(4):
        s, sn = SCALES[i], SCALES[i + 1]
        wh = [
            (s, s),
            (math.sqrt(s * sn), math.sqrt(s * sn)),
            (s * math.sqrt(2.0), s / math.sqrt(2.0)),
            (s * math.sqrt(0.5), s / math.sqrt(0.5)),
        ]
        for k in range(NB):
            base = 14 * k
            tab[i, base + 8] = wh[k][0]
            tab[i, base + 9] = wh[k][1]
            tab[i, base + 10:base + 14] = (0.1, 0.1, 0.2, 0.2)
    return tab


_CONST56 = _anchor_const_table()


def _ssd_head_kernel(f0, f1, f2, f3, w0, w1, w2, w3, btab, ctab, out_ref,
                     scratch):
    j = pl.program_id(1)
    feats = (f0, f1, f2, f3)
    ws = (w0, w1, w2, w3)

    for i in range(4):
        fw = FHW[i]
        npix = NPIX[i]
        start = BLK_START[i]
        in_level = (j >= start) & (j < start + NBLK[i])

        @pl.when(in_level & (j == start))
        def _conv(i=i, fw=fw, npix=npix):
            fh = fw
            w = ws[i][...]
            bias = jnp.broadcast_to(btab[i:i + 1, 0:NCH], (fw, NCH))
            nchunk = 4 if i == 0 else 1
            cr = fh // nchunk
            for c in range(nchunk):
                r0, r1 = c * cr, (c + 1) * cr
                lo, hi = max(0, r0 - 1), min(fh, r1 + 1)
                x = feats[i][0, lo:hi].reshape((hi - lo) * fw, CH[i])
                t = jnp.dot(x, w, preferred_element_type=jnp.float32)
                t3 = t.reshape(hi - lo, fw, 9 * NCH)
                scratch[r0:r1, 0:fw, :] = jnp.broadcast_to(
                    bias[None], (r1 - r0, fw, NCH))
                for ky in range(3):
                    for kx in range(3):
                        oy, ox = ky - 1, kx - 1
                        q = ky * 3 + kx
                        y0 = max(r0, -oy)
                        y1 = min(r1, fh - oy)
                        x0 = max(0, -ox)
                        x1 = fw - max(0, ox)
                        term = t3[y0 + oy - lo:y1 + oy - lo,
                                  x0 + ox:x1 + ox,
                                  q * NCH:(q + 1) * NCH]
                        scratch[y0:y1, x0:x1, :] += term

        @pl.when(in_level)
        def _emit(i=i, fw=fw, start=start):
            lb = j - start
            nr = PBLK // fw
            rows = scratch[pl.ds(lb * nr, nr), 0:fw, :].reshape(PBLK, NCH)
            conf = rows[:, 0:NCONF]
            locv = rows[:, NCONF:NCH]
            lane = jax.lax.broadcasted_iota(jnp.int32, (PBLK, NCONF), 1)
            swapped = jnp.where(jnp.bitwise_and(lane, 1) == 0,
                                jnp.roll(conf, -1, axis=1),
                                jnp.roll(conf, 1, axis=1))
            p8 = jax.nn.sigmoid(conf - swapped)
            pix = lb * PBLK + jax.lax.broadcasted_iota(jnp.int32, (PBLK, 1), 0)
            xcol = jnp.bitwise_and(pix, fw - 1)
            yrow = jax.lax.shift_right_logical(pix, int(math.log2(fw)))
            scale = float(STEPS[i]) / float(IMG)
            cx = (xcol.astype(jnp.float32) + 0.5) * scale
            cy = (yrow.astype(jnp.float32) + 0.5) * scale
            pieces = []
            for k in range(NB):
                b = 14 * k
                pieces.append(p8[:, 2 * k:2 * k + 2])
                pieces.append(locv[:, 4 * k:4 * k + 4])
                pieces.append(cx)
                pieces.append(cy)
                pieces.append(jnp.broadcast_to(ctab[i:i + 1, b + 8:b + 14],
                                               (PBLK, 6)))
            out_ref[0] = jnp.concatenate(pieces, axis=1)


def kernel(feat0, feat1, feat2, feat3, Wc0, bc0, Wl0, bl0, Wc1, bc1, Wl1,
           bl1, Wc2, bc2, Wl2, bl2, Wc3, bc3, Wl3, bl3):
    B = feat0.shape[0]
    feats = (feat0, feat1, feat2, feat3)
    Wc = (Wc0, Wc1, Wc2, Wc3)
    bc = (bc0, bc1, bc2, bc3)
    Wl = (Wl0, Wl1, Wl2, Wl3)
    bl = (bl0, bl1, bl2, bl3)

    w_all = []
    for i in range(4):
        blocks = []
        for ky in range(3):
            for kx in range(3):
                blocks.append(Wc[i][ky, kx])
                blocks.append(Wl[i][ky, kx])
        w_all.append(jnp.concatenate(blocks, axis=-1))

    btab = jnp.zeros((8, 128), jnp.float32)
    for i in range(4):
        btab = btab.at[i, 0:NCONF].set(bc[i])
        btab = btab.at[i, NCONF:NCH].set(bl[i])
    ctab = jnp.zeros((8, 128), jnp.float32)
    ctab = ctab.at[0:4, 0:NB * 14].set(jnp.asarray(_CONST56))

    def feat_spec(i):
        s = BLK_START[i]
        return pl.BlockSpec(
            (1, FHW[i], FHW[i], CH[i]),
            lambda b, j, s=s: (jnp.minimum(b + (j > s).astype(jnp.int32),
                                           B - 1), 0, 0, 0))

    def whole(arr):
        return pl.BlockSpec(arr.shape, lambda b, j: (0,) * arr.ndim)

    out = pl.pallas_call(
        _ssd_head_kernel,
        grid=(B, NBLK_ALL),
        in_specs=[feat_spec(i) for i in range(4)]
        + [whole(w) for w in w_all] + [whole(btab), whole(ctab)],
        out_specs=pl.BlockSpec((1, PBLK, NB * 14), lambda b, j: (b, j, 0)),
        out_shape=jax.ShapeDtypeStruct((B, NPIX_ALL, NB * 14), jnp.float32),
        scratch_shapes=[pltpu.VMEM((FHW[0], FHW[0], NCH), jnp.float32)],
        compiler_params=pltpu.CompilerParams(
            dimension_semantics=("arbitrary", "arbitrary"),
            vmem_limit_bytes=128 * 1024 * 1024,
        ),
    )(*feats, *w_all, btab, ctab)
    return out.reshape(B, NPIX_ALL * NB, 14)
